# baseline (speedup 1.0000x reference)
import functools

import jax
import jax.numpy as jnp
import numpy as np
from jax import lax
from jax.experimental import pallas as pl
from jax.experimental.pallas import tpu as pltpu
from jax.experimental.pallas import tpu_sc as plsc

_VOCAB = 100000
_MAX_LEN = 2048
_D = 768
_B = 4

_NC = 2
_NS = 16
_NW = _NC * _NS
_P = _MAX_LEN // _NW
_LANES = 16
_CPR = _D // _LANES


def _pos_encoding_np(max_len: int, d_model: int) -> np.ndarray:
    pos = np.arange(max_len, dtype=np.float32)[:, None]
    two_i = np.arange(0, d_model, 2, dtype=np.float32)
    ang = pos / (np.float32(10000.0) ** (two_i / np.float32(d_model)))
    enc = np.zeros((max_len, d_model), dtype=np.float32)
    enc[:, 0::2] = np.sin(ang)
    enc[:, 1::2] = np.cos(ang)
    return enc


_ENC = _pos_encoding_np(_MAX_LEN, _D)


_S = 32
_NCHUNK = _B * _P // _S
_NBUF = 3
_AHEAD = 2


def _sc_body(x_hbm, table_hbm, enc_hbm, out_hbm, idx_v, enc_v,
             buf0, buf1, buf2, idx_sem, enc_sem,
             g0, g1, g2, s0, s1, s2):
    c = lax.axis_index("c")
    s = lax.axis_index("s")
    w = s * _NC + c
    bufs = (buf0, buf1, buf2)
    gsem = (g0, g1, g2)
    ssem = (s0, s1, s2)
    cpb = _P // _S

    idescs = [pltpu.async_copy(x_hbm.at[b, pl.ds(w * _P, _P)],
                               idx_v.at[b], idx_sem) for b in range(_B)]
    edesc = pltpu.async_copy(enc_hbm.at[pl.ds(w * _P, _P)], enc_v, enc_sem)
    for d in idescs:
        d.wait()

    gdesc = [None] * _NCHUNK
    sdesc = [None] * _NCHUNK

    def fire_gather(i):
        j = i % _NBUF
        if i >= _NBUF:
            sdesc[i - _NBUF].wait()
        b, h = i // cpb, i % cpb
        gdesc[i] = pltpu.async_copy(
            table_hbm.at[idx_v.at[b, pl.ds(h * _S, _S)]], bufs[j], gsem[j])

    for i in range(_AHEAD):
        fire_gather(i)
    edesc.wait()
    for i in range(_NCHUNK):
        j = i % _NBUF
        b, h = i // cpb, i % cpb
        with jax.named_scope(f"gwait{i}"):
            gdesc[i].wait()
        if i + _AHEAD < _NCHUNK:
            with jax.named_scope(f"gfire{i}"):
                fire_gather(i + _AHEAD)
        buf = bufs[j]

        with jax.named_scope(f"add{i}"):
            @pl.loop(0, _S)
            def _row_add(r):
                for cc in range(_CPR):
                    sl = pl.ds(cc * _LANES, _LANES)
                    plsc.addupdate(buf.at[r, sl], enc_v[h * _S + r, sl])

        with jax.named_scope(f"sfire{i}"):
            sdesc[i] = pltpu.async_copy(
                buf, out_hbm.at[pl.ds(b * _MAX_LEN + w * _P + h * _S, _S)],
                ssem[j])
    for i in range(_NCHUNK - _NBUF, _NCHUNK):
        sdesc[i].wait()


@functools.partial(jax.jit, static_argnames=())
def kernel(x, table):
    x32 = x.astype(jnp.int32)
    enc = jnp.asarray(_ENC)
    mesh = plsc.VectorSubcoreMesh(core_axis_name="c", subcore_axis_name="s")
    out = pl.kernel(
        _sc_body,
        out_type=jax.ShapeDtypeStruct((_B * _MAX_LEN, _D), jnp.float32),
        mesh=mesh,
        scratch_types=[
            pltpu.VMEM((_B, _P), jnp.int32),
            pltpu.VMEM((_P, _D), jnp.float32),
        ] + [pltpu.VMEM((_S, _D), jnp.float32)] * _NBUF
          + [pltpu.SemaphoreType.DMA] * (2 + 2 * _NBUF),
    )(x32, table, enc)
    return out.reshape(_B, _MAX_LEN, _D)


# speedup vs baseline: 1.2219x; 1.2219x over previous
import functools

import jax
import jax.numpy as jnp
import numpy as np
from jax import lax
from jax.experimental import pallas as pl
from jax.experimental.pallas import tpu as pltpu
from jax.experimental.pallas import tpu_sc as plsc

_VOCAB = 100000
_MAX_LEN = 2048
_D = 768
_B = 4

_NC = 2
_NS = 16
_NW = _NC * _NS
_P = _MAX_LEN // _NW
_LANES = 16
_CPR = _D // _LANES


def _pos_encoding_np(max_len: int, d_model: int) -> np.ndarray:
    pos = np.arange(max_len, dtype=np.float32)[:, None]
    two_i = np.arange(0, d_model, 2, dtype=np.float32)
    ang = pos / (np.float32(10000.0) ** (two_i / np.float32(d_model)))
    enc = np.zeros((max_len, d_model), dtype=np.float32)
    enc[:, 0::2] = np.sin(ang)
    enc[:, 1::2] = np.cos(ang)
    return enc


_ENC = _pos_encoding_np(_MAX_LEN, _D)


_S = 8
_NG = _P // _S
_NRING = 4
_AHEAD = 3


def _sc_body(x_hbm, table_hbm, enc_hbm, out_hbm, idx_v,
             e0, e1, e2, e3, r0, r1, r2, r3, idx_sem,
             g0, g1, g2, g3, s0, s1, s2, s3):
    c = lax.axis_index("c")
    s = lax.axis_index("s")
    w = s * _NC + c
    encb = (e0, e1, e2, e3)
    rows = (r0, r1, r2, r3)
    gsem = (g0, g1, g2, g3)
    ssem = (s0, s1, s2, s3)

    idescs = [pltpu.async_copy(x_hbm.at[b, pl.ds(w * _P, _P)],
                               idx_v.at[b], idx_sem) for b in range(_B)]
    for d in idescs:
        d.wait()

    gdesc = [None] * _NG
    sdesc = [None] * _NG

    def fire_group(q):
        grp = q % _NRING
        if q >= _NRING:
            for d in sdesc[q - _NRING]:
                d.wait()
        ge = pltpu.async_copy(
            enc_hbm.at[pl.ds(w * _P + q * _S, _S)], encb[grp], gsem[grp])
        gr = [pltpu.async_copy(
            table_hbm.at[idx_v.at[b, pl.ds(q * _S, _S)]],
            rows[grp].at[b], gsem[grp]) for b in range(_B)]
        gdesc[q] = [ge] + gr

    for q in range(_AHEAD):
        fire_group(q)
    for q in range(_NG):
        grp = q % _NRING
        with jax.named_scope(f"gwait{q}"):
            for d in gdesc[q]:
                d.wait()
        if q + _AHEAD < _NG:
            with jax.named_scope(f"gfire{q}"):
                fire_group(q + _AHEAD)
        eb, rb = encb[grp], rows[grp]

        with jax.named_scope(f"add{q}"):
            @pl.loop(0, _S)
            def _row_add(r):
                for cc in range(_CPR):
                    sl = pl.ds(cc * _LANES, _LANES)
                    v = eb[r, sl]
                    for b in range(_B):
                        plsc.addupdate(rb.at[b, r, sl], v)

        with jax.named_scope(f"sfire{q}"):
            sdesc[q] = [pltpu.async_copy(
                rb.at[b],
                out_hbm.at[pl.ds(b * _MAX_LEN + w * _P + q * _S, _S)],
                ssem[grp]) for b in range(_B)]
    for q in range(_NG - _NRING, _NG):
        for d in sdesc[q]:
            d.wait()


@functools.partial(jax.jit, static_argnames=())
def kernel(x, table):
    x32 = x.astype(jnp.int32)
    enc = jnp.asarray(_ENC)
    mesh = plsc.VectorSubcoreMesh(core_axis_name="c", subcore_axis_name="s")
    out = pl.kernel(
        _sc_body,
        out_type=jax.ShapeDtypeStruct((_B * _MAX_LEN, _D), jnp.float32),
        mesh=mesh,
        scratch_types=[
            pltpu.VMEM((_B, _P), jnp.int32),
        ] + [pltpu.VMEM((_S, _D), jnp.float32)] * _NRING
          + [pltpu.VMEM((_B, _S, _D), jnp.float32)] * _NRING
          + [pltpu.SemaphoreType.DMA] * (1 + 2 * _NRING),
    )(x32, table, enc)
    return out.reshape(_B, _MAX_LEN, _D)
